# Initial kernel scaffold; baseline (speedup 1.0000x reference)
import jax
import jax.numpy as jnp
from jax.experimental import pallas as pl


def kernel(entity_table, relation_table, W_R, h, r, pos_t, neg_t):
    raise NotImplementedError("write your pallas kernel here")


# speedup vs baseline: 6.9038x; 6.9038x over previous
import functools

import jax
import jax.numpy as jnp
from jax import lax
from jax.experimental import pallas as pl
from jax.experimental.pallas import tpu as pltpu
from jax.experimental.pallas import tpu_sc as plsc

N_ENT = 100000
N_REL = 16
D_IN = 128
D_REL = 64
M = 16384
REG_KG = 0.01

NC, NS = 2, 16
NW = NC * NS
TOTAL = 3 * M
CHUNK = 128
N_CHUNK = TOTAL // (NW * CHUNK)


def _sc_gather_body(idx_hbm, table_hbm, out_hbm, idx_v, buf0, buf1, sem0, sem1):
    wid = lax.axis_index("s") * NC + lax.axis_index("c")
    rows = N_CHUNK * CHUNK
    base = wid * rows
    pltpu.sync_copy(idx_hbm.at[pl.ds(base, rows)], idx_v)
    bufs = (buf0, buf1)
    sems = (sem0, sem1)
    handles = [None] * N_CHUNK
    handles[0] = pltpu.async_copy(
        table_hbm.at[idx_v.at[pl.ds(0, CHUNK)]], buf0, sem0)
    for j in range(N_CHUNK):
        if j + 1 < N_CHUNK:
            handles[j + 1] = pltpu.async_copy(
                table_hbm.at[idx_v.at[pl.ds((j + 1) * CHUNK, CHUNK)]],
                bufs[(j + 1) % 2], sems[(j + 1) % 2])
        handles[j].wait()
        pltpu.sync_copy(bufs[j % 2],
                        out_hbm.at[pl.ds(base + j * CHUNK, CHUNK)])


@functools.cache
def _sc_gather():
    return pl.kernel(
        _sc_gather_body,
        mesh=plsc.VectorSubcoreMesh(core_axis_name="c", subcore_axis_name="s"),
        out_type=jax.ShapeDtypeStruct((TOTAL, D_IN), jnp.float32),
        scratch_types=[
            pltpu.VMEM((N_CHUNK * CHUNK,), jnp.int32),
            pltpu.VMEM((CHUNK, D_IN), jnp.float32),
            pltpu.VMEM((CHUNK, D_IN), jnp.float32),
            pltpu.SemaphoreType.DMA,
            pltpu.SemaphoreType.DMA,
        ],
    )


BLK = 1024
GRID = M // BLK
NCOL = N_REL * D_REL


def _normalize(x):
    n2 = jnp.sum(x * x, axis=1, keepdims=True)
    return x / jnp.maximum(jnp.sqrt(n2), 1e-12)


def _tc_body(gath_ref, r_ref, wall_ref, rel_ref, out_ref):
    i = pl.program_id(0)
    r_col = r_ref[...]
    wall = wall_ref[...]

    col_rel = lax.broadcasted_iota(jnp.int32, (BLK, NCOL), 1) // D_REL
    colmask = col_rel == r_col

    def project(x):
        y = lax.dot_general(x, wall, (((1,), (0,)), ((), ())),
                            preferred_element_type=jnp.float32)
        y = jnp.where(colmask, y, 0.0)
        acc = y[:, 0:D_REL]
        for k in range(1, N_REL):
            acc = acc + y[:, k * D_REL:(k + 1) * D_REL]
        return acc

    h_vec = _normalize(project(gath_ref[0]))
    pos_t_vec = _normalize(project(gath_ref[1]))
    neg_t_vec = _normalize(project(gath_ref[2]))

    onehot = (r_col == lax.broadcasted_iota(jnp.int32, (BLK, N_REL), 1)
              ).astype(jnp.float32)
    r_vec = _normalize(lax.dot_general(onehot, rel_ref[...],
                                       (((1,), (0,)), ((), ())),
                                       preferred_element_type=jnp.float32))

    d_pos = h_vec + r_vec - pos_t_vec
    d_neg = h_vec + r_vec - neg_t_vec
    pos_score = jnp.sum(d_pos * d_pos, axis=1, keepdims=True)
    neg_score = jnp.sum(d_neg * d_neg, axis=1, keepdims=True)
    z = neg_score - pos_score
    li = jnp.maximum(-z, 0.0) + jnp.log(1.0 + jnp.exp(-jnp.abs(z)))

    reg = 0.5 * (jnp.sum(h_vec * h_vec) + jnp.sum(r_vec * r_vec)
                 + jnp.sum(pos_t_vec * pos_t_vec)
                 + jnp.sum(neg_t_vec * neg_t_vec))
    partial = (jnp.sum(li) + REG_KG * reg).reshape(1, 1)

    acc = jnp.where(i == 0, partial, out_ref[...] + partial)
    out_ref[...] = jnp.where(i == GRID - 1, acc * (1.0 / M), acc)


_tc_compute = pl.pallas_call(
    _tc_body,
    grid=(GRID,),
    in_specs=[
        pl.BlockSpec((3, BLK, D_IN), lambda i: (0, i, 0)),
        pl.BlockSpec((BLK, 1), lambda i: (i, 0)),
        pl.BlockSpec((D_IN, NCOL), lambda i: (0, 0)),
        pl.BlockSpec((N_REL, D_REL), lambda i: (0, 0)),
    ],
    out_specs=pl.BlockSpec((1, 1), lambda i: (0, 0)),
    out_shape=jax.ShapeDtypeStruct((1, 1), jnp.float32),
)


def kernel(entity_table, relation_table, W_R, h, r, pos_t, neg_t):
    idx = jnp.concatenate([h, pos_t, neg_t]).astype(jnp.int32)
    gathered = _sc_gather()(idx, entity_table)
    gath3 = gathered.reshape(3, M, D_IN)
    wall = jnp.transpose(W_R, (1, 0, 2)).reshape(D_IN, NCOL)
    r_col = r.astype(jnp.int32).reshape(M, 1)
    out = _tc_compute(gath3, r_col, wall, relation_table)
    return out[0, 0]


# speedup vs baseline: 8.0228x; 1.1621x over previous
import functools

import jax
import jax.numpy as jnp
from jax import lax
from jax.experimental import pallas as pl
from jax.experimental.pallas import tpu as pltpu
from jax.experimental.pallas import tpu_sc as plsc

N_ENT = 100000
N_REL = 16
D_IN = 128
D_REL = 64
M = 16384
REG_KG = 0.01

NSPLIT = 2
H = M // NSPLIT

NC, NS = 2, 16
NW = NC * NS
CHUNK = 128


def _sc_gather_body(n_chunk, idx_hbm, table_hbm, out_hbm,
                    idx_v, buf0, buf1, sem0, sem1):
    wid = lax.axis_index("s") * NC + lax.axis_index("c")
    rows = n_chunk * CHUNK
    base = wid * rows
    pltpu.sync_copy(idx_hbm.at[pl.ds(base, rows)], idx_v)
    bufs = (buf0, buf1)
    sems = (sem0, sem1)
    handles = [None] * n_chunk
    handles[0] = pltpu.async_copy(
        table_hbm.at[idx_v.at[pl.ds(0, CHUNK)]], buf0, sem0)
    for j in range(n_chunk):
        if j + 1 < n_chunk:
            handles[j + 1] = pltpu.async_copy(
                table_hbm.at[idx_v.at[pl.ds((j + 1) * CHUNK, CHUNK)]],
                bufs[(j + 1) % 2], sems[(j + 1) % 2])
        handles[j].wait()
        pltpu.sync_copy(bufs[j % 2],
                        out_hbm.at[pl.ds(base + j * CHUNK, CHUNK)])


@functools.cache
def _sc_gather(total_rows):
    n_chunk = total_rows // (NW * CHUNK)
    return pl.kernel(
        functools.partial(_sc_gather_body, n_chunk),
        mesh=plsc.VectorSubcoreMesh(core_axis_name="c", subcore_axis_name="s"),
        out_type=jax.ShapeDtypeStruct((total_rows, D_IN), jnp.float32),
        scratch_types=[
            pltpu.VMEM((n_chunk * CHUNK,), jnp.int32),
            pltpu.VMEM((CHUNK, D_IN), jnp.float32),
            pltpu.VMEM((CHUNK, D_IN), jnp.float32),
            pltpu.SemaphoreType.DMA,
            pltpu.SemaphoreType.DMA,
        ],
    )


BLK = 1024
NCOL = N_REL * D_REL


def _normalize(x):
    n2 = jnp.sum(x * x, axis=1, keepdims=True)
    return x / jnp.maximum(jnp.sqrt(n2), 1e-12)


def _tc_body(grid, gath_ref, r_ref, wall_ref, rel_ref, g_ref, out_ref):
    i = pl.program_id(0)
    r_col = r_ref[...]
    wall = wall_ref[...].astype(jnp.bfloat16)
    g = g_ref[...]

    col_rel = lax.broadcasted_iota(jnp.int32, (BLK, NCOL), 1) // D_REL
    colmask = col_rel == r_col

    def project(x):
        y = lax.dot_general(x.astype(jnp.bfloat16), wall,
                            (((1,), (0,)), ((), ())),
                            preferred_element_type=jnp.float32)
        y = jnp.where(colmask, y, 0.0).astype(jnp.bfloat16)
        return lax.dot_general(y, g, (((1,), (0,)), ((), ())),
                               preferred_element_type=jnp.float32)

    h_vec = _normalize(project(gath_ref[0]))
    pos_t_vec = _normalize(project(gath_ref[1]))
    neg_t_vec = _normalize(project(gath_ref[2]))

    onehot = (r_col == lax.broadcasted_iota(jnp.int32, (BLK, N_REL), 1)
              ).astype(jnp.float32)
    r_vec = _normalize(lax.dot_general(onehot, rel_ref[...],
                                       (((1,), (0,)), ((), ())),
                                       preferred_element_type=jnp.float32))

    d_pos = h_vec + r_vec - pos_t_vec
    d_neg = h_vec + r_vec - neg_t_vec
    pos_score = jnp.sum(d_pos * d_pos, axis=1, keepdims=True)
    neg_score = jnp.sum(d_neg * d_neg, axis=1, keepdims=True)
    z = neg_score - pos_score
    li = jnp.maximum(-z, 0.0) + jnp.log(1.0 + jnp.exp(-jnp.abs(z)))

    reg = 0.5 * (jnp.sum(h_vec * h_vec) + jnp.sum(r_vec * r_vec)
                 + jnp.sum(pos_t_vec * pos_t_vec)
                 + jnp.sum(neg_t_vec * neg_t_vec))
    partial = ((jnp.sum(li) + REG_KG * reg) * (1.0 / M)).reshape(1, 1)

    acc = jnp.where(i == 0, partial, out_ref[...] + partial)
    out_ref[...] = acc


@functools.cache
def _tc_compute(rows):
    grid = rows // BLK
    return pl.pallas_call(
        functools.partial(_tc_body, grid),
        grid=(grid,),
        in_specs=[
            pl.BlockSpec((3, BLK, D_IN), lambda i: (0, i, 0)),
            pl.BlockSpec((BLK, 1), lambda i: (i, 0)),
            pl.BlockSpec((D_IN, NCOL), lambda i: (0, 0)),
            pl.BlockSpec((N_REL, D_REL), lambda i: (0, 0)),
            pl.BlockSpec((NCOL, D_REL), lambda i: (0, 0)),
        ],
        out_specs=pl.BlockSpec((1, 1), lambda i: (0, 0)),
        out_shape=jax.ShapeDtypeStruct((1, 1), jnp.float32),
    )


def kernel(entity_table, relation_table, W_R, h, r, pos_t, neg_t):
    h = h.astype(jnp.int32)
    r = r.astype(jnp.int32)
    pos_t = pos_t.astype(jnp.int32)
    neg_t = neg_t.astype(jnp.int32)
    wall = jnp.transpose(W_R, (1, 0, 2)).reshape(D_IN, NCOL)
    g = (jnp.arange(NCOL, dtype=jnp.int32)[:, None] % D_REL
         == jnp.arange(D_REL, dtype=jnp.int32)[None, :]).astype(jnp.bfloat16)

    parts = []
    for p in range(NSPLIT):
        lo = p * H
        idx = jnp.concatenate([h[lo:lo + H], pos_t[lo:lo + H],
                               neg_t[lo:lo + H]])
        gathered = _sc_gather(3 * H)(idx, entity_table)
        parts.append(gathered.reshape(3, H, D_IN))

    out = None
    for p in range(NSPLIT):
        r_col = r[p * H:(p + 1) * H].reshape(H, 1)
        o = _tc_compute(H)(parts[p], r_col, wall, relation_table, g)
        out = o if out is None else out + o
    return out[0, 0]
